# baseline (speedup 1.0000x reference)
import functools

import jax
import jax.numpy as jnp
from jax import lax
from jax.experimental import pallas as pl
from jax.experimental.pallas import tpu as pltpu
from jax.experimental.pallas import tpu_sc as plsc

NUM_EXPERTS_K = 64
TOP_K_K = 8
HIDDEN_K = 2048
TOKENS_K = 32768
BLOCK_T = 2048

_INFO = plsc.get_sparse_core_info()
_NC, _NS = _INFO.num_cores, _INFO.num_subcores
_NW = _NC * _NS
_CHUNK = TOKENS_K // _NW
_GROUPS = _CHUNK // 16


def _mm_block(x_ref, wt_ref, logits_ref, lt_ref):
    x = x_ref[...]
    wt = wt_ref[...]
    logits = jnp.dot(x, wt, preferred_element_type=jnp.float32)
    logits_ref[...] = logits
    lt_ref[...] = logits.T


_sc_mesh = plsc.VectorSubcoreMesh(core_axis_name="c", subcore_axis_name="s")


@functools.partial(
    pl.kernel,
    mesh=_sc_mesh,
    out_type=[
        jax.ShapeDtypeStruct((TOP_K_K, TOKENS_K), jnp.float32),
        jax.ShapeDtypeStruct((TOP_K_K, TOKENS_K), jnp.int32),
    ],
    scratch_types=[
        pltpu.VMEM((NUM_EXPERTS_K, _CHUNK), jnp.float32),
        pltpu.VMEM((TOP_K_K, _CHUNK), jnp.float32),
        pltpu.VMEM((TOP_K_K, _CHUNK), jnp.int32),
    ],
)
def _sc_topk(lt_hbm, topv_hbm, topi_hbm, chunk_v, outv_v, outi_v):
    wid = lax.axis_index("s") * _NC + lax.axis_index("c")
    base = wid * _CHUNK
    pltpu.sync_copy(lt_hbm.at[:, pl.ds(base, _CHUNK)], chunk_v)

    def group_body(g, _):
        def insert(e, carry):
            ts = list(carry[:TOP_K_K])
            js = list(carry[TOP_K_K:])
            c = chunk_v[e, pl.ds(g * 16, 16)]
            ci = jnp.full((16,), 0, jnp.int32) + e
            for k in range(TOP_K_K):
                gt = c > ts[k]
                tk = jnp.where(gt, c, ts[k])
                c = jnp.where(gt, ts[k], c)
                jk = jnp.where(gt, ci, js[k])
                ci = jnp.where(gt, js[k], ci)
                ts[k], js[k] = tk, jk
            return (*ts, *js)

        init = tuple(
            [jnp.full((16,), -jnp.inf, jnp.float32)] * TOP_K_K
            + [jnp.full((16,), 0, jnp.int32)] * TOP_K_K
        )
        carry = lax.fori_loop(0, NUM_EXPERTS_K, insert, init)
        ts = carry[:TOP_K_K]
        js = carry[TOP_K_K:]
        es = [jnp.exp(t - ts[0]) for t in ts]
        total = es[0]
        for k in range(1, TOP_K_K):
            total = total + es[k]
        r = 1.0 / total
        for k in range(TOP_K_K):
            outv_v[k, pl.ds(g * 16, 16)] = es[k] * r
            outi_v[k, pl.ds(g * 16, 16)] = js[k]
        return 0

    lax.fori_loop(0, _GROUPS, group_body, 0)
    pltpu.sync_copy(outv_v, topv_hbm.at[:, pl.ds(base, _CHUNK)])
    pltpu.sync_copy(outi_v, topi_hbm.at[:, pl.ds(base, _CHUNK)])


@jax.jit
def kernel(hidden_states, weight):
    wt = weight.T
    grid = (TOKENS_K // BLOCK_T,)
    logits, lt = pl.pallas_call(
        _mm_block,
        grid=grid,
        in_specs=[
            pl.BlockSpec((BLOCK_T, HIDDEN_K), lambda i: (i, 0)),
            pl.BlockSpec((HIDDEN_K, NUM_EXPERTS_K), lambda i: (0, 0)),
        ],
        out_specs=[
            pl.BlockSpec((BLOCK_T, NUM_EXPERTS_K), lambda i: (i, 0)),
            pl.BlockSpec((NUM_EXPERTS_K, BLOCK_T), lambda i: (0, i)),
        ],
        out_shape=[
            jax.ShapeDtypeStruct((TOKENS_K, NUM_EXPERTS_K), jnp.float32),
            jax.ShapeDtypeStruct((NUM_EXPERTS_K, TOKENS_K), jnp.float32),
        ],
        compiler_params=pltpu.CompilerParams(
            dimension_semantics=("parallel",),
        ),
    )(hidden_states, wt)
    topv_t, topi_t = _sc_topk(lt)
    return logits, topv_t.T, topi_t.T


# speedup vs baseline: 1.0102x; 1.0102x over previous
import functools

import jax
import jax.numpy as jnp
from jax import lax
from jax.experimental import pallas as pl
from jax.experimental.pallas import tpu as pltpu
from jax.experimental.pallas import tpu_sc as plsc

NUM_EXPERTS_K = 64
TOP_K_K = 8
HIDDEN_K = 2048
TOKENS_K = 32768
BLOCK_T = 2048

_INFO = plsc.get_sparse_core_info()
_NC, _NS = _INFO.num_cores, _INFO.num_subcores
_NW = _NC * _NS
_CHUNK = TOKENS_K // _NW
_GROUPS = _CHUNK // 16


def _mm_block(x_ref, wt_ref, logits_ref, lt_ref):
    x = x_ref[...]
    wt = wt_ref[...]
    logits = jnp.dot(x, wt, preferred_element_type=jnp.float32)
    logits_ref[...] = logits
    lt_ref[...] = logits.T


_sc_mesh = plsc.VectorSubcoreMesh(core_axis_name="c", subcore_axis_name="s")


@functools.partial(
    pl.kernel,
    mesh=_sc_mesh,
    out_type=[
        jax.ShapeDtypeStruct((TOP_K_K, TOKENS_K), jnp.float32),
        jax.ShapeDtypeStruct((TOP_K_K, TOKENS_K), jnp.int32),
    ],
    scratch_types=[
        pltpu.VMEM((NUM_EXPERTS_K, _CHUNK), jnp.float32),
        pltpu.VMEM((TOP_K_K, _CHUNK), jnp.float32),
        pltpu.VMEM((TOP_K_K, _CHUNK), jnp.int32),
    ],
)
def _sc_topk(lt_hbm, topv_hbm, topi_hbm, chunk_v, outv_v, outi_v):
    wid = lax.axis_index("s") * _NC + lax.axis_index("c")
    base = wid * _CHUNK
    pltpu.sync_copy(lt_hbm.at[:, pl.ds(base, _CHUNK)], chunk_v)

    def group_body(g, _):
        def insert(e, carry):
            n = 2 * TOP_K_K
            ts = [list(carry[0:TOP_K_K]), list(carry[n:n + TOP_K_K])]
            js = [list(carry[TOP_K_K:n]), list(carry[n + TOP_K_K:2 * n])]
            for h in range(2):
                c = chunk_v[e, pl.ds((2 * g + h) * 16, 16)]
                ci = jnp.full((16,), 0, jnp.int32) + e
                for k in range(TOP_K_K):
                    gt = c > ts[h][k]
                    tk = jnp.where(gt, c, ts[h][k])
                    c = jnp.where(gt, ts[h][k], c)
                    jk = jnp.where(gt, ci, js[h][k])
                    ci = jnp.where(gt, js[h][k], ci)
                    ts[h][k], js[h][k] = tk, jk
            return (*ts[0], *js[0], *ts[1], *js[1])

        init_half = (
            [jnp.full((16,), -jnp.inf, jnp.float32)] * TOP_K_K
            + [jnp.full((16,), 0, jnp.int32)] * TOP_K_K
        )
        carry = lax.fori_loop(0, NUM_EXPERTS_K, insert, tuple(init_half * 2))
        n = 2 * TOP_K_K
        for h in range(2):
            ts = carry[h * n:h * n + TOP_K_K]
            js = carry[h * n + TOP_K_K:(h + 1) * n]
            es = [jnp.exp(t - ts[0]) for t in ts]
            total = es[0]
            for k in range(1, TOP_K_K):
                total = total + es[k]
            r = 1.0 / total
            for k in range(TOP_K_K):
                outv_v[k, pl.ds((2 * g + h) * 16, 16)] = es[k] * r
                outi_v[k, pl.ds((2 * g + h) * 16, 16)] = js[k]
        return 0

    lax.fori_loop(0, _GROUPS // 2, group_body, 0)
    pltpu.sync_copy(outv_v, topv_hbm.at[:, pl.ds(base, _CHUNK)])
    pltpu.sync_copy(outi_v, topi_hbm.at[:, pl.ds(base, _CHUNK)])


@jax.jit
def kernel(hidden_states, weight):
    wt = weight.T
    grid = (TOKENS_K // BLOCK_T,)
    logits, lt = pl.pallas_call(
        _mm_block,
        grid=grid,
        in_specs=[
            pl.BlockSpec((BLOCK_T, HIDDEN_K), lambda i: (i, 0)),
            pl.BlockSpec((HIDDEN_K, NUM_EXPERTS_K), lambda i: (0, 0)),
        ],
        out_specs=[
            pl.BlockSpec((BLOCK_T, NUM_EXPERTS_K), lambda i: (i, 0)),
            pl.BlockSpec((NUM_EXPERTS_K, BLOCK_T), lambda i: (0, i)),
        ],
        out_shape=[
            jax.ShapeDtypeStruct((TOKENS_K, NUM_EXPERTS_K), jnp.float32),
            jax.ShapeDtypeStruct((NUM_EXPERTS_K, TOKENS_K), jnp.float32),
        ],
        compiler_params=pltpu.CompilerParams(
            dimension_semantics=("parallel",),
        ),
    )(hidden_states, wt)
    topv_t, topi_t = _sc_topk(lt)
    return logits, topv_t.T, topi_t.T
